# Initial kernel scaffold; baseline (speedup 1.0000x reference)
import jax
import jax.numpy as jnp
from jax.experimental import pallas as pl


def kernel(mem, val, idx):
    raise NotImplementedError("write your pallas kernel here")


# speedup vs baseline: 3.7903x; 3.7903x over previous
import functools

import jax
import jax.numpy as jnp
from jax import lax
from jax.experimental import pallas as pl
from jax.experimental.pallas import tpu as pltpu
from jax.experimental.pallas import tpu_sc as plsc

T = 8
B = 16384
D = 64
NC = 2
NS = 16
NW = NC * NS
TAG = 1 << 20
SLOTS = TAG // NW
ROWS = B // NW
CH = 128


@functools.cache
def _mesh():
    return plsc.VectorSubcoreMesh(
        core_axis_name="c", subcore_axis_name="s",
        num_cores=NC, num_subcores=NS)


def _rate_body(valT_ref, out_ref):
    v = valT_ref[...]
    n = jnp.round(v * float(T)).astype(jnp.int32)
    nz = (n != 0) & (n != T)
    nsafe = jnp.clip(n, 1, None).astype(jnp.float32)
    spacing = float(T) / nsafe
    acc = jnp.zeros_like(v)
    for c in range(T):
        cond = nz & (jnp.floor(c / spacing) < nsafe) & (
            jnp.floor(jnp.mod(float(c), spacing)) == 0.0)
        acc = acc + jnp.where(n == T, 1.0, cond.astype(jnp.float32))
    out_ref[:, 0:D] = acc.T * (1.0 / float(T))


def _rate(valT):
    blk = B // 8
    return pl.pallas_call(
        _rate_body,
        out_shape=jax.ShapeDtypeStruct((B, 2 * D), jnp.float32),
        grid=(8,),
        in_specs=[pl.BlockSpec((D, blk), lambda i: (0, i))],
        out_specs=pl.BlockSpec((blk, 2 * D), lambda i: (i, 0)),
    )(valT)


def _relayout_body(memT_ref, out_ref):
    out_ref[:, 0:D] = memT_ref[...].T


def _relayout(memT):
    m = memT.shape[1]
    blk = 2048
    grid = (m + blk - 1) // blk
    return pl.pallas_call(
        _relayout_body,
        out_shape=jax.ShapeDtypeStruct((m, 2 * D), jnp.float32),
        grid=(grid,),
        in_specs=[pl.BlockSpec((D, blk), lambda i: (0, i))],
        out_specs=pl.BlockSpec((blk, 2 * D), lambda i: (i, 0)),
    )(memT)


def _xlane(x, i):
    return lax.gather(
        x, i[:, None],
        lax.GatherDimensionNumbers(
            offset_dims=(), collapsed_slice_dims=(0,), start_index_map=(0,)),
        (1,),
        mode=lax.GatherScatterMode.PROMISE_IN_BOUNDS)


def _tag_body(idx_hbm, tag_hbm, idx_v, tag_v):
    wid = lax.axis_index("s") * NC + lax.axis_index("c")
    pltpu.sync_copy(idx_hbm, idx_v)
    lanes = lax.iota(jnp.int32, 16)

    def body(k, carry):
        x = idx_v[pl.ds(k * 16, 16)]
        dead = lanes < 0
        for s in range(1, 16):
            shifted = _xlane(x, jnp.minimum(lanes + s, 15))
            dead = dead | ((x == shifted) & (lanes < (16 - s)))
        keep = ((x >> 15) == wid) & jnp.logical_not(dead)
        ival = k * 16 + lanes
        plsc.store_scatter(tag_v, [x & (SLOTS - 1)], ival, mask=keep)
        return carry

    lax.fori_loop(0, B // 16, body, 0)
    pltpu.sync_copy(tag_v, tag_hbm.at[pl.ds(wid * SLOTS, SLOTS)])


@functools.cache
def _tag_kernel():
    return functools.partial(
        pl.kernel,
        out_type=jax.ShapeDtypeStruct((TAG,), jnp.int32),
        mesh=_mesh(),
        scratch_types=[
            pltpu.VMEM((B,), jnp.int32),
            pltpu.VMEM((SLOTS,), jnp.int32),
        ],
        compiler_params=pltpu.CompilerParams(needs_layout_passes=False),
    )(_tag_body)


def _asm_body(memP_hbm, rate_hbm, idx_hbm, tag_hbm, outT_hbm,
              idx_v, w_v, mrows, rrows, obuf, sem):
    wid = lax.axis_index("s") * NC + lax.axis_index("c")
    base = wid * ROWS
    pltpu.sync_copy(idx_hbm.at[pl.ds(base, ROWS)], idx_v)
    for c in range(ROWS // CH):
        sl = pl.ds(c * CH, CH)
        pltpu.async_copy(tag_hbm.at[idx_v.at[sl]], w_v.at[sl], sem).wait()

    lanes = lax.iota(jnp.int32, 16)

    for c in range(ROWS // CH):
        sl = pl.ds(c * CH, CH)
        pltpu.async_copy(memP_hbm.at[idx_v.at[sl]], mrows, sem).wait()
        pltpu.async_copy(rate_hbm.at[w_v.at[sl]], rrows, sem).wait()

        def chunk_body(il, carry):
            col = jnp.full((16,), c * CH + il, jnp.int32)
            for j in range(D // 16):
                mv = mrows[il, pl.ds(j * 16, 16)]
                rv = rrows[il, pl.ds(j * 16, 16)]
                plsc.store_scatter(obuf, [lanes + j * 16, col], mv + rv)
            return carry

        lax.fori_loop(0, CH, chunk_body, 0)

    pltpu.sync_copy(obuf, outT_hbm.at[:, pl.ds(base, ROWS)])


@functools.cache
def _asm_kernel():
    return functools.partial(
        pl.kernel,
        out_type=jax.ShapeDtypeStruct((D, B), jnp.float32),
        mesh=_mesh(),
        scratch_types=[
            pltpu.VMEM((ROWS,), jnp.int32),
            pltpu.VMEM((ROWS,), jnp.int32),
            pltpu.VMEM((CH, 2 * D), jnp.float32),
            pltpu.VMEM((CH, 2 * D), jnp.float32),
            pltpu.VMEM((D, ROWS), jnp.float32),
            pltpu.SemaphoreType.DMA,
        ],
        compiler_params=pltpu.CompilerParams(needs_layout_passes=False),
    )(_asm_body)


def kernel(mem, val, idx):
    idx32 = idx.astype(jnp.int32)
    memT = mem.T
    valT = val.T
    memP = _relayout(memT)
    rateP = _rate(valT)
    tag = _tag_kernel()(idx32)
    outT = _asm_kernel()(memP, rateP, idx32, tag)
    return outT.T


# speedup vs baseline: 3.8209x; 1.0081x over previous
import functools

import jax
import jax.numpy as jnp
from jax import lax
from jax.experimental import pallas as pl
from jax.experimental.pallas import tpu as pltpu
from jax.experimental.pallas import tpu_sc as plsc

T = 8
B = 16384
D = 64
NC = 2
NS = 16
NW = NC * NS
TAG = 1 << 20
SLOTS = TAG // NW
ROWS = B // NW
CH = 128


@functools.cache
def _mesh():
    return plsc.VectorSubcoreMesh(
        core_axis_name="c", subcore_axis_name="s",
        num_cores=NC, num_subcores=NS)


def _rate_body(valT_ref, out_ref):
    v = valT_ref[...]
    n = jnp.round(v * float(T)).astype(jnp.int32)
    nz = (n != 0) & (n != T)
    nsafe = jnp.clip(n, 1, None).astype(jnp.float32)
    spacing = float(T) / nsafe
    acc = jnp.zeros_like(v)
    for c in range(T):
        cond = nz & (jnp.floor(c / spacing) < nsafe) & (
            jnp.floor(jnp.mod(float(c), spacing)) == 0.0)
        acc = acc + jnp.where(n == T, 1.0, cond.astype(jnp.float32))
    out_ref[:, 0:D] = acc.T * (1.0 / float(T))


def _rate(valT):
    blk = B // 8
    return pl.pallas_call(
        _rate_body,
        out_shape=jax.ShapeDtypeStruct((B, 2 * D), jnp.float32),
        grid=(8,),
        in_specs=[pl.BlockSpec((D, blk), lambda i: (0, i))],
        out_specs=pl.BlockSpec((blk, 2 * D), lambda i: (i, 0)),
    )(valT)


def _relayout_body(memT_ref, out_ref):
    blk = memT_ref[...]
    out_ref[:, 0:D] = blk[:, 0:1024].T
    out_ref[:, D:2 * D] = blk[:, 1024:2048].T


def _relayout(memT):
    m = memT.shape[1]
    blk = 2048
    grid = (m + blk - 1) // blk
    return pl.pallas_call(
        _relayout_body,
        out_shape=jax.ShapeDtypeStruct((grid * 1024, 2 * D), jnp.float32),
        grid=(grid,),
        in_specs=[pl.BlockSpec((D, blk), lambda i: (0, i))],
        out_specs=pl.BlockSpec((blk // 2, 2 * D), lambda i: (i, 0)),
    )(memT)


def _xlane(x, i):
    return lax.gather(
        x, i[:, None],
        lax.GatherDimensionNumbers(
            offset_dims=(), collapsed_slice_dims=(0,), start_index_map=(0,)),
        (1,),
        mode=lax.GatherScatterMode.PROMISE_IN_BOUNDS)


def _tag_body(idx_hbm, tag_hbm, idx_v, tag_v):
    wid = lax.axis_index("s") * NC + lax.axis_index("c")
    pltpu.sync_copy(idx_hbm, idx_v)
    lanes = lax.iota(jnp.int32, 16)

    def body(k, carry):
        x = idx_v[pl.ds(k * 16, 16)]
        dead = lanes < 0
        for s in range(1, 16):
            shifted = _xlane(x, jnp.minimum(lanes + s, 15))
            dead = dead | ((x == shifted) & (lanes < (16 - s)))
        keep = ((x >> 15) == wid) & jnp.logical_not(dead)
        ival = k * 16 + lanes
        plsc.store_scatter(tag_v, [x & (SLOTS - 1)], ival, mask=keep)
        return carry

    lax.fori_loop(0, B // 16, body, 0)
    pltpu.sync_copy(tag_v, tag_hbm.at[pl.ds(wid * SLOTS, SLOTS)])


@functools.cache
def _tag_kernel():
    return functools.partial(
        pl.kernel,
        out_type=jax.ShapeDtypeStruct((TAG,), jnp.int32),
        mesh=_mesh(),
        scratch_types=[
            pltpu.VMEM((B,), jnp.int32),
            pltpu.VMEM((SLOTS,), jnp.int32),
        ],
        compiler_params=pltpu.CompilerParams(needs_layout_passes=False),
    )(_tag_body)


def _asm_body(memP_hbm, rate_hbm, idx_hbm, tag_hbm, outT_hbm,
              idx_v, hrow_v, w_v, mrows, rrows, obuf, sem):
    wid = lax.axis_index("s") * NC + lax.axis_index("c")
    base = wid * ROWS
    pltpu.sync_copy(idx_hbm.at[pl.ds(base, ROWS)], idx_v)

    def half_body(k, carry):
        sl = pl.ds(k * 16, 16)
        x = idx_v[sl]
        hrow_v[sl] = ((x >> 11) << 10) | (x & 1023)
        return carry

    lax.fori_loop(0, ROWS // 16, half_body, 0)

    for c in range(ROWS // CH):
        sl = pl.ds(c * CH, CH)
        pltpu.async_copy(tag_hbm.at[idx_v.at[sl]], w_v.at[sl], sem).wait()

    lanes = lax.iota(jnp.int32, 16)

    for c in range(ROWS // CH):
        sl = pl.ds(c * CH, CH)
        pltpu.async_copy(memP_hbm.at[hrow_v.at[sl]], mrows, sem).wait()
        pltpu.async_copy(rate_hbm.at[w_v.at[sl]], rrows, sem).wait()

        def chunk_body(g, carry):
            pv = (idx_v[pl.ds(c * CH + g * 16, 16)] >> 10) & 1
            for l in range(16):
                il = g * 16 + l
                p64 = _xlane(pv, jnp.full((16,), l, jnp.int32)) * D
                col = jnp.full((16,), c * CH + il, jnp.int32)
                for j in range(2 * D // 16):
                    tgt = lanes + j * 16 - p64
                    msk = (tgt >= 0) & (tgt < D)
                    mv = mrows[il, pl.ds(j * 16, 16)]
                    rv = rrows[il, pl.ds((j % (D // 16)) * 16, 16)]
                    plsc.store_scatter(
                        obuf, [tgt & (D - 1), col], mv + rv, mask=msk)
            return carry

        lax.fori_loop(0, CH // 16, chunk_body, 0)

    pltpu.sync_copy(obuf, outT_hbm.at[:, pl.ds(base, ROWS)])


@functools.cache
def _asm_kernel():
    return functools.partial(
        pl.kernel,
        out_type=jax.ShapeDtypeStruct((D, B), jnp.float32),
        mesh=_mesh(),
        scratch_types=[
            pltpu.VMEM((ROWS,), jnp.int32),
            pltpu.VMEM((ROWS,), jnp.int32),
            pltpu.VMEM((ROWS,), jnp.int32),
            pltpu.VMEM((CH, 2 * D), jnp.float32),
            pltpu.VMEM((CH, 2 * D), jnp.float32),
            pltpu.VMEM((D, ROWS), jnp.float32),
            pltpu.SemaphoreType.DMA,
        ],
        compiler_params=pltpu.CompilerParams(needs_layout_passes=False),
    )(_asm_body)


def kernel(mem, val, idx):
    idx32 = idx.astype(jnp.int32)
    memT = mem.T
    valT = val.T
    memP = _relayout(memT)
    rateP = _rate(valT)
    tag = _tag_kernel()(idx32)
    outT = _asm_kernel()(memP, rateP, idx32, tag)
    return outT.T


# speedup vs baseline: 4.3360x; 1.1348x over previous
import functools

import jax
import jax.numpy as jnp
from jax import lax
from jax.experimental import pallas as pl
from jax.experimental.pallas import tpu as pltpu
from jax.experimental.pallas import tpu_sc as plsc

T = 8
B = 16384
D = 64
M = 1000000
NC = 2
NS = 16
NW = NC * NS
WIN = 512
WPT = 61
RPT = WPT * WIN
STUB = 1953 * WIN
LCAP = 640


@functools.cache
def _mesh():
    return plsc.VectorSubcoreMesh(
        core_axis_name="c", subcore_axis_name="s",
        num_cores=NC, num_subcores=NS)


def _rate_body(valT_ref, out_ref):
    v = valT_ref[...]
    n = jnp.round(v * float(T)).astype(jnp.int32)
    nz = (n != 0) & (n != T)
    nsafe = jnp.clip(n, 1, None).astype(jnp.float32)
    spacing = float(T) / nsafe
    acc = jnp.zeros_like(v)
    for c in range(T):
        cond = nz & (jnp.floor(c / spacing) < nsafe) & (
            jnp.floor(jnp.mod(float(c), spacing)) == 0.0)
        acc = acc + jnp.where(n == T, 1.0, cond.astype(jnp.float32))
    out_ref[:, 0:D] = acc.T * (1.0 / float(T))


def _rate(valT):
    return pl.pallas_call(
        _rate_body,
        out_shape=jax.ShapeDtypeStruct((B, 2 * D), jnp.float32),
        grid=(8,),
        in_specs=[pl.BlockSpec((D, B // 8), lambda i: (0, i))],
        out_specs=pl.BlockSpec((B // 8, 2 * D), lambda i: (i, 0)),
    )(valT)


def _untranspose_body(outP_ref, out_ref):
    out_ref[...] = outP_ref[...][:, 0:D].T


def _untranspose(outP):
    return pl.pallas_call(
        _untranspose_body,
        out_shape=jax.ShapeDtypeStruct((D, B), jnp.float32),
        grid=(8,),
        in_specs=[pl.BlockSpec((B // 8, 2 * D), lambda i: (i, 0))],
        out_specs=pl.BlockSpec((D, B // 8), lambda i: (0, i)),
    )(outP)


def _xlane(x, i):
    return lax.gather(
        x, i[:, None],
        lax.GatherDimensionNumbers(
            offset_dims=(), collapsed_slice_dims=(0,), start_index_map=(0,)),
        (1,),
        mode=lax.GatherScatterMode.PROMISE_IN_BOUNDS)


def _flow_body(memT_hbm, rateP_hbm, idx_hbm, stubT_hbm, outP_hbm,
               ibuf, ilist, rlist, wlist, tagl, cbuf, rrflat, wbuf, sbuf,
               staging, oidx, sem):
    cid = lax.axis_index("c")
    sid = lax.axis_index("s")
    wid = sid * NC + cid
    lanes = lax.iota(jnp.int32, 16)
    lo = wid * RPT
    hi = jnp.where(wid == NW - 1, M, lo + RPT)
    lane0 = lanes == 0

    def scan_chunk(cc, cnt):
        pltpu.sync_copy(idx_hbm.at[pl.ds(cc * 2048, 2048)], ibuf)

        def scan_vreg(k, cnt):
            x = ibuf[pl.ds(k * 16, 16)]
            m = (x >= lo) & (x < hi)

            def cond(st):
                m, _ = st
                return plsc.all_reduce_population_count(m)[0] > 0

            def body(st):
                m, cnt = st
                lsp = plsc.all_reduce_ffs(m)
                rsp = _xlane(x, lsp)
                isp = cc * 2048 + k * 16 + lsp
                cix = jnp.full((16,), cnt, jnp.int32)
                plsc.store_scatter(ilist, [cix], isp, mask=lane0)
                plsc.store_scatter(rlist, [cix], rsp, mask=lane0)
                plsc.store_scatter(tagl, [rsp - lo], isp, mask=lane0)
                return m & (lanes != lsp), cnt + 1

            m, cnt = lax.while_loop(cond, body, (m, cnt))
            return cnt

        return lax.fori_loop(0, 128, scan_vreg, cnt)

    cnt = lax.fori_loop(0, B // 2048, scan_chunk, 0)

    def wv_body(v, carry):
        valid = (v * 16 + lanes) < cnt
        rv = rlist[pl.ds(v * 16, 16)]
        wv = plsc.load_gather(tagl, [rv - lo], mask=valid)
        wlist[pl.ds(v * 16, 16)] = jnp.where(valid, wv, 0)
        return carry

    lax.fori_loop(0, LCAP // 16, wv_body, 0)

    for c in range(LCAP // D):
        pltpu.async_copy(
            rateP_hbm.at[wlist.at[pl.ds(c * D, D)]], cbuf, sem).wait()

        def cp_body(k, carry):
            ksp = jnp.full((16,), k, jnp.int32)
            for j in range(D // 16):
                cj = lanes + j * 16
                v = plsc.load_gather(cbuf, [ksp, cj])
                plsc.store_scatter(
                    rrflat, [(c * D + k) * D + cj], v)
            return carry

        lax.fori_loop(0, D, cp_body, 0)

    oidx[...] = B + lanes

    def process_window(buf, wbase, wwidth, p):
        def scan_mem(v, p):
            valid = (v * 16 + lanes) < cnt2
            rv = rlist[pl.ds(v * 16, 16)]
            m = valid & (rv >= wbase) & (rv < wbase + wwidth)

            def cond(st):
                m, _ = st
                return plsc.all_reduce_population_count(m)[0] > 0

            def body(st):
                m, p = st
                lsp = plsc.all_reduce_ffs(m)
                rsp = _xlane(rv, lsp)
                isp = _xlane(ilist[pl.ds(v * 16, 16)], lsp)
                col = rsp - wbase
                mo = jnp.full((16,), v * 16, jnp.int32) + lsp

                @pl.when(p[0] == 16)
                def _():
                    pltpu.sync_copy(staging, outP_hbm.at[oidx])
                    oidx[...] = B + lanes

                p = jnp.where(p == 16, 0, p)
                pv = jnp.full((16,), 0, jnp.int32) + p
                for j in range(D // 16):
                    cj = lanes + j * 16
                    mv = plsc.load_gather(buf, [cj, col])
                    rvv = plsc.load_gather(rrflat, [mo * D + cj])
                    plsc.store_scatter(staging, [pv, cj], mv + rvv)
                plsc.store_scatter(oidx, [pv], isp, mask=lane0)
                return m & (lanes != lsp), p + 1

            m, p = lax.while_loop(cond, body, (m, p))
            return p

        return lax.fori_loop(0, LCAP // 16, scan_mem, p)

    cnt2 = cnt
    nwin = jnp.where(wid == NW - 1, WPT + 1, WPT)

    def win_body(win, p):
        wbase = pl.multiple_of(lo + win * WIN, WIN)
        pltpu.sync_copy(memT_hbm.at[:, pl.ds(wbase, WIN)], wbuf)
        return process_window(wbuf, wbase, WIN, p)

    p = lax.fori_loop(0, nwin, win_body, jnp.full((16,), 0, jnp.int32))

    @pl.when(wid == NW - 1)
    def _():
        pltpu.sync_copy(stubT_hbm, sbuf)

    p = lax.cond(
        wid == NW - 1,
        lambda p: process_window(sbuf, STUB, M - STUB, p),
        lambda p: p, p)

    pltpu.sync_copy(staging, outP_hbm.at[oidx])


@functools.cache
def _flow_kernel():
    return functools.partial(
        pl.kernel,
        out_type=jax.ShapeDtypeStruct((B + 16, 2 * D), jnp.float32),
        mesh=_mesh(),
        scratch_types=[
            pltpu.VMEM((2048,), jnp.int32),
            pltpu.VMEM((LCAP,), jnp.int32),
            pltpu.VMEM((LCAP,), jnp.int32),
            pltpu.VMEM((LCAP,), jnp.int32),
            pltpu.VMEM((M - (NW - 1) * RPT,), jnp.int32),
            pltpu.VMEM((D, 2 * D), jnp.float32),
            pltpu.VMEM((LCAP * D,), jnp.float32),
            pltpu.VMEM((D, WIN), jnp.float32),
            pltpu.VMEM((D, M - STUB), jnp.float32),
            pltpu.VMEM((16, 2 * D), jnp.float32),
            pltpu.VMEM((16,), jnp.int32),
            pltpu.SemaphoreType.DMA,
        ],
        compiler_params=pltpu.CompilerParams(needs_layout_passes=False),
    )(_flow_body)


def kernel(mem, val, idx):
    idx32 = idx.astype(jnp.int32)
    memT = mem.T
    valT = val.T
    rateP = _rate(valT)
    stubT = memT[:, STUB:]
    outP = _flow_kernel()(memT, rateP, idx32, stubT)
    return _untranspose(outP).T


# speedup vs baseline: 4.7880x; 1.1042x over previous
import functools

import jax
import jax.numpy as jnp
from jax import lax
from jax.experimental import pallas as pl
from jax.experimental.pallas import tpu as pltpu
from jax.experimental.pallas import tpu_sc as plsc

T = 8
B = 16384
D = 64
M = 1000000
NC = 2
NS = 16
NW = NC * NS
WIN = 256
WPT = 122
RPT = WPT * WIN
STUB = 3906 * WIN
LCAP = 640


@functools.cache
def _mesh():
    return plsc.VectorSubcoreMesh(
        core_axis_name="c", subcore_axis_name="s",
        num_cores=NC, num_subcores=NS)


def _rate_body(valT_ref, out_ref):
    v = valT_ref[...]
    n = jnp.round(v * float(T)).astype(jnp.int32)
    nz = (n != 0) & (n != T)
    nsafe = jnp.clip(n, 1, None).astype(jnp.float32)
    spacing = float(T) / nsafe
    acc = jnp.zeros_like(v)
    for c in range(T):
        cond = nz & (jnp.floor(c / spacing) < nsafe) & (
            jnp.floor(jnp.mod(float(c), spacing)) == 0.0)
        acc = acc + jnp.where(n == T, 1.0, cond.astype(jnp.float32))
    out_ref[:, 0:D] = acc.T * (1.0 / float(T))


def _rate(valT):
    return pl.pallas_call(
        _rate_body,
        out_shape=jax.ShapeDtypeStruct((B, 2 * D), jnp.float32),
        grid=(8,),
        in_specs=[pl.BlockSpec((D, B // 8), lambda i: (0, i))],
        out_specs=pl.BlockSpec((B // 8, 2 * D), lambda i: (i, 0)),
    )(valT)


def _untranspose_body(outP_ref, out_ref):
    out_ref[...] = outP_ref[...][:, 0:D].T


def _untranspose(outP):
    return pl.pallas_call(
        _untranspose_body,
        out_shape=jax.ShapeDtypeStruct((D, B), jnp.float32),
        grid=(8,),
        in_specs=[pl.BlockSpec((B // 8, 2 * D), lambda i: (i, 0))],
        out_specs=pl.BlockSpec((D, B // 8), lambda i: (0, i)),
    )(outP)


def _xlane(x, i):
    return lax.gather(
        x, i[:, None],
        lax.GatherDimensionNumbers(
            offset_dims=(), collapsed_slice_dims=(0,), start_index_map=(0,)),
        (1,),
        mode=lax.GatherScatterMode.PROMISE_IN_BOUNDS)


def _flow_body(memT_hbm, rateP_hbm, idx_hbm, stubT_hbm, outP_hbm,
               ibuf, ilist, rlist, wlist, tagl, cbuf, rrflat, wbuf, wbuf2,
               sbuf, staging, oidx, sem, sem2):
    cid = lax.axis_index("c")
    sid = lax.axis_index("s")
    wid = sid * NC + cid
    lanes = lax.iota(jnp.int32, 16)
    lo = wid * RPT
    hi = jnp.where(wid == NW - 1, M, lo + RPT)
    lane0 = lanes == 0

    def scan_chunk(cc, cnt):
        pltpu.sync_copy(idx_hbm.at[pl.ds(cc * 2048, 2048)], ibuf)

        def scan_vreg(k, cnt):
            x = ibuf[pl.ds(k * 16, 16)]
            m = (x >= lo) & (x < hi)

            def cond(st):
                m, _ = st
                return plsc.all_reduce_population_count(m)[0] > 0

            def body(st):
                m, cnt = st
                lsp = plsc.all_reduce_ffs(m)
                rsp = _xlane(x, lsp)
                isp = cc * 2048 + k * 16 + lsp
                cix = jnp.full((16,), cnt, jnp.int32)
                plsc.store_scatter(ilist, [cix], isp, mask=lane0)
                plsc.store_scatter(rlist, [cix], rsp, mask=lane0)
                plsc.store_scatter(tagl, [rsp - lo], isp, mask=lane0)
                return m & (lanes != lsp), cnt + 1

            m, cnt = lax.while_loop(cond, body, (m, cnt))
            return cnt

        return lax.fori_loop(0, 128, scan_vreg, cnt)

    cnt = lax.fori_loop(0, B // 2048, scan_chunk, 0)

    def wv_body(v, carry):
        valid = (v * 16 + lanes) < cnt
        rv = rlist[pl.ds(v * 16, 16)]
        wv = plsc.load_gather(tagl, [rv - lo], mask=valid)
        wlist[pl.ds(v * 16, 16)] = jnp.where(valid, wv, 0)
        return carry

    lax.fori_loop(0, LCAP // 16, wv_body, 0)

    for c in range(LCAP // D):
        pltpu.async_copy(
            rateP_hbm.at[wlist.at[pl.ds(c * D, D)]], cbuf, sem).wait()

        def cp_body(k, carry):
            ksp = jnp.full((16,), k, jnp.int32)
            for j in range(D // 16):
                cj = lanes + j * 16
                v = plsc.load_gather(cbuf, [ksp, cj])
                plsc.store_scatter(
                    rrflat, [(c * D + k) * D + cj], v)
            return carry

        lax.fori_loop(0, D, cp_body, 0)

    oidx[...] = B + lanes

    def process_window(buf, wbase, wwidth, p):
        def scan_mem(v, p):
            valid = (v * 16 + lanes) < cnt2
            rv = rlist[pl.ds(v * 16, 16)]
            m = valid & (rv >= wbase) & (rv < wbase + wwidth)

            def cond(st):
                m, _ = st
                return plsc.all_reduce_population_count(m)[0] > 0

            def body(st):
                m, p = st
                lsp = plsc.all_reduce_ffs(m)
                rsp = _xlane(rv, lsp)
                isp = _xlane(ilist[pl.ds(v * 16, 16)], lsp)
                col = rsp - wbase
                mo = jnp.full((16,), v * 16, jnp.int32) + lsp

                @pl.when(p[0] == 16)
                def _():
                    pltpu.sync_copy(staging, outP_hbm.at[oidx])
                    oidx[...] = B + lanes

                p = jnp.where(p == 16, 0, p)
                pv = jnp.full((16,), 0, jnp.int32) + p
                for j in range(D // 16):
                    cj = lanes + j * 16
                    mv = plsc.load_gather(buf, [cj, col])
                    rvv = plsc.load_gather(rrflat, [mo * D + cj])
                    plsc.store_scatter(staging, [pv, cj], mv + rvv)
                plsc.store_scatter(oidx, [pv], isp, mask=lane0)
                return m & (lanes != lsp), p + 1

            m, p = lax.while_loop(cond, body, (m, p))
            return p

        return lax.fori_loop(0, LCAP // 16, scan_mem, p)

    cnt2 = cnt
    npair = jnp.where(wid == NW - 1, (WPT + 2) // 2, WPT // 2)
    wbufs = (wbuf, wbuf2)
    sems = (sem, sem2)

    def src_at(win):
        wb = pl.multiple_of(
            jnp.minimum(lo + win * WIN, (STUB // WIN - 1) * WIN), WIN)
        return memT_hbm.at[:, pl.ds(wb, WIN)]

    for b in range(2):
        pltpu.async_copy(src_at(b), wbufs[b], sems[b])

    def pair_body(q, p):
        for b in range(2):
            win = q * 2 + b
            wbase = pl.multiple_of(lo + win * WIN, WIN)
            pltpu.make_async_copy(src_at(win), wbufs[b], sems[b]).wait()
            p = process_window(wbufs[b], wbase, WIN, p)
            pltpu.async_copy(src_at(win + 2), wbufs[b], sems[b])
        return p

    p = lax.fori_loop(0, npair, pair_body, jnp.full((16,), 0, jnp.int32))
    for b in range(2):
        pltpu.make_async_copy(src_at(0), wbufs[b], sems[b]).wait()

    @pl.when(wid == NW - 1)
    def _():
        pltpu.sync_copy(stubT_hbm, sbuf)

    p = lax.cond(
        wid == NW - 1,
        lambda p: process_window(sbuf, STUB, M - STUB, p),
        lambda p: p, p)

    pltpu.sync_copy(staging, outP_hbm.at[oidx])


@functools.cache
def _flow_kernel():
    return functools.partial(
        pl.kernel,
        out_type=jax.ShapeDtypeStruct((B + 16, 2 * D), jnp.float32),
        mesh=_mesh(),
        scratch_types=[
            pltpu.VMEM((2048,), jnp.int32),
            pltpu.VMEM((LCAP,), jnp.int32),
            pltpu.VMEM((LCAP,), jnp.int32),
            pltpu.VMEM((LCAP,), jnp.int32),
            pltpu.VMEM((M - (NW - 1) * RPT,), jnp.int32),
            pltpu.VMEM((D, 2 * D), jnp.float32),
            pltpu.VMEM((LCAP * D,), jnp.float32),
            pltpu.VMEM((D, WIN), jnp.float32),
            pltpu.VMEM((D, WIN), jnp.float32),
            pltpu.VMEM((D, M - STUB), jnp.float32),
            pltpu.VMEM((16, 2 * D), jnp.float32),
            pltpu.VMEM((16,), jnp.int32),
            pltpu.SemaphoreType.DMA,
            pltpu.SemaphoreType.DMA,
        ],
        compiler_params=pltpu.CompilerParams(needs_layout_passes=False),
    )(_flow_body)


def kernel(mem, val, idx):
    idx32 = idx.astype(jnp.int32)
    memT = mem.T
    valT = val.T
    rateP = _rate(valT)
    stubT = memT[:, STUB:]
    outP = _flow_kernel()(memT, rateP, idx32, stubT)
    return _untranspose(outP).T
